# baseline (speedup 1.0000x reference)
import functools

import jax
import jax.numpy as jnp
from jax import lax
from jax.experimental import pallas as pl
from jax.experimental.pallas import tpu as pltpu
from jax.experimental.pallas import tpu_sc as plsc

E = 8
H = 1024
I = 2048
T = 2048
K = 2

BLK = 272
NBMAX = (K * T) // BLK + E - 1
PMAX = NBMAX * BLK

NC = 2
NS = 16
NW = NC * NS
TPW = T // NW
HT = TPW // 2


def _dot_t(a, b):
    return jax.lax.dot_general(
        a, b, (((1,), (1,)), ((), ())), preferred_element_type=jnp.float32
    )


def _router_body(
    x_ref, rw_ref, logits_ref, p0_ref, p1_ref, c0_ref, c1_ref, bexp_ref, nvalid_ref
):
    x = x_ref[...]
    logits = _dot_t(x, rw_ref[...])
    logits_ref[...] = logits
    m = jnp.max(logits, axis=1, keepdims=True)
    ex = jnp.exp(logits - m)
    probs = ex / jnp.sum(ex, axis=1, keepdims=True)
    eidx = jax.lax.broadcasted_iota(jnp.int32, (T, E), 1)
    m1 = jnp.max(probs, axis=1, keepdims=True)
    i1 = jnp.min(jnp.where(probs == m1, eidx, E), axis=1, keepdims=True)
    probs2 = jnp.where(eidx == i1, -1.0, probs)
    m2 = jnp.max(probs2, axis=1, keepdims=True)
    i2 = jnp.min(jnp.where(probs2 == m2, eidx, E), axis=1, keepdims=True)
    denom = m1 + m2
    c0_ref[...] = jnp.broadcast_to(m1 / denom, (T, 16))
    c1_ref[...] = jnp.broadcast_to(m2 / denom, (T, 16))

    onehot0 = (eidx == i1).astype(jnp.float32)
    onehot1 = (eidx == i2).astype(jnp.float32)
    oh = jnp.concatenate([onehot0, onehot1], axis=1)
    CH = 512
    rr = jax.lax.broadcasted_iota(jnp.int32, (CH, CH), 0)
    cc = jax.lax.broadcasted_iota(jnp.int32, (CH, CH), 1)
    tric = (rr > cc).astype(jnp.float32)
    carry = jnp.zeros((1, 2 * E), jnp.float32)
    chunks = []
    for k in range(T // CH):
        ohk = oh[k * CH : (k + 1) * CH]
        chunks.append(
            jnp.dot(tric, ohk, preferred_element_type=jnp.float32) + carry
        )
        carry = carry + jnp.sum(ohk, axis=0, keepdims=True)
    pref = jnp.concatenate(chunks, axis=0)
    pref0 = pref[:, :E]
    pref1 = pref[:, E:]
    counts0 = carry[:, :E]
    counts = counts0 + carry[:, E:]
    padded = jnp.floor((counts + (BLK - 1)) * (1.0 / BLK)) * BLK
    eci = jax.lax.broadcasted_iota(jnp.int32, (E, E), 0)
    ecj = jax.lax.broadcasted_iota(jnp.int32, (E, E), 1)
    off = jnp.dot(
        padded, (eci < ecj).astype(jnp.float32), preferred_element_type=jnp.float32
    )
    nb = padded * (1.0 / BLK)
    cnb = jnp.dot(
        nb, (eci <= ecj).astype(jnp.float32), preferred_element_type=jnp.float32
    )

    def sel(tab, idx):
        return jnp.sum(jnp.where(eidx == idx, tab, 0.0), axis=1, keepdims=True)

    pos0 = sel(off, i1) + sel(pref0, i1)
    pos1 = sel(off, i2) + sel(counts0, i2) + sel(pref1, i2)
    p0_ref[...] = pos0.astype(jnp.int32)
    p1_ref[...] = pos1.astype(jnp.int32)

    bi = jax.lax.broadcasted_iota(jnp.int32, (1, NBMAX), 1).astype(jnp.float32)
    bexp = jnp.zeros((1, NBMAX), jnp.float32)
    for e in range(E):
        bexp += (bi >= cnb[0:1, e : e + 1]).astype(jnp.float32)
    bexp_ref[...] = bexp.astype(jnp.int32)
    nvalid_ref[...] = cnb[0:1, E - 1 : E].astype(jnp.int32)


def _router(x, router_w):
    return pl.pallas_call(
        _router_body,
        out_shape=(
            jax.ShapeDtypeStruct((T, E), jnp.float32),
            jax.ShapeDtypeStruct((T, 1), jnp.int32),
            jax.ShapeDtypeStruct((T, 1), jnp.int32),
            jax.ShapeDtypeStruct((T, 16), jnp.float32),
            jax.ShapeDtypeStruct((T, 16), jnp.float32),
            jax.ShapeDtypeStruct((1, NBMAX), jnp.int32),
            jax.ShapeDtypeStruct((1, 1), jnp.int32),
        ),
    )(x, router_w)


@functools.cache
def _sc_dispatch_kernel():
    mesh = plsc.VectorSubcoreMesh(
        core_axis_name="c", subcore_axis_name="s", num_cores=NC, num_subcores=NS
    )

    @functools.partial(
        pl.kernel,
        out_type=jax.ShapeDtypeStruct((PMAX, H), jnp.float32),
        mesh=mesh,
        scratch_types=[
            pltpu.VMEM((TPW,), jnp.int32),
            pltpu.VMEM((TPW,), jnp.int32),
            pltpu.VMEM((TPW, H), jnp.float32),
            pltpu.SemaphoreType.DMA,
            pltpu.SemaphoreType.DMA,
        ],
    )
    def dispatch(x_hbm, p0_hbm, p1_hbm, xs_hbm, idx0_v, idx1_v, rows_v, sem0, sem1):
        wid = lax.axis_index("s") * NC + lax.axis_index("c")
        base = wid * TPW
        pltpu.sync_copy(p0_hbm.at[wid], idx0_v)
        pltpu.sync_copy(p1_hbm.at[wid], idx1_v)
        pltpu.sync_copy(x_hbm.at[pl.ds(base, TPW)], rows_v)
        h0 = pltpu.async_copy(rows_v, xs_hbm.at[idx0_v], sem0)
        h1 = pltpu.async_copy(rows_v, xs_hbm.at[idx1_v], sem1)
        h0.wait()
        h1.wait()

    return dispatch


def _sc_dispatch(x, p0w, p1w):
    return _sc_dispatch_kernel()(x, p0w, p1w)


def _grouped_body(
    bexp_s, nvalid_s, xs_ref, w_in_ref, b_in_ref, w_out_ref, b_out_ref, out_ref
):
    i = pl.program_id(0)
    e = jnp.minimum(bexp_s[i], E - 1)

    @pl.when(i < nvalid_s[0])
    def _():
        x = xs_ref[...]
        mid = _dot_t(x, w_in_ref[0])
        mid = mid + b_in_ref[e, :][None, :]
        mid = 0.5 * mid * (1.0 + jax.lax.erf(mid * 0.7071067811865476))
        y = _dot_t(mid, w_out_ref[0])
        out_ref[...] = y + b_out_ref[e, :][None, :]


def _grouped(bexp, nvalid, xs, w_in, b_in, w_out, b_out):
    def emap(i, bexp_s, nvalid_s):
        return (jnp.minimum(bexp_s[i], E - 1), 0, 0)

    grid_spec = pltpu.PrefetchScalarGridSpec(
        num_scalar_prefetch=2,
        grid=(NBMAX,),
        in_specs=[
            pl.BlockSpec((BLK, H), lambda i, b, n: (i, 0)),
            pl.BlockSpec((1, I, H), emap),
            pl.BlockSpec((E, I), lambda i, b, n: (0, 0)),
            pl.BlockSpec((1, H, I), emap),
            pl.BlockSpec((E, H), lambda i, b, n: (0, 0)),
        ],
        out_specs=pl.BlockSpec((BLK, H), lambda i, b, n: (i, 0)),
    )
    return pl.pallas_call(
        _grouped_body,
        grid_spec=grid_spec,
        out_shape=jax.ShapeDtypeStruct((PMAX, H), jnp.float32),
        compiler_params=pltpu.CompilerParams(
            dimension_semantics=("arbitrary",),
        ),
    )(bexp, nvalid, xs, w_in, b_in, w_out, b_out)


@functools.cache
def _sc_combine_kernel():
    mesh = plsc.VectorSubcoreMesh(
        core_axis_name="c", subcore_axis_name="s", num_cores=NC, num_subcores=NS
    )

    @functools.partial(
        pl.kernel,
        out_type=jax.ShapeDtypeStruct((T, H), jnp.float32),
        mesh=mesh,
        scratch_types=[
            pltpu.VMEM((HT,), jnp.int32),
            pltpu.VMEM((HT,), jnp.int32),
            pltpu.VMEM((HT, H), jnp.float32),
            pltpu.VMEM((HT, H), jnp.float32),
            pltpu.VMEM((TPW, 16), jnp.float32),
            pltpu.VMEM((TPW, 16), jnp.float32),
            pltpu.SemaphoreType.DMA,
            pltpu.SemaphoreType.DMA,
        ],
    )
    def combine(
        ys_hbm, p0_hbm, p1_hbm, c0_hbm, c1_hbm, out_hbm,
        i0_v, i1_v, r0_v, r1_v, c0_v, c1_v, s0, s1,
    ):
        wid = lax.axis_index("s") * NC + lax.axis_index("c")
        base = wid * TPW
        pltpu.sync_copy(c0_hbm.at[wid], c0_v)
        pltpu.sync_copy(c1_hbm.at[wid], c1_v)
        for h in range(TPW // HT):
            hb = h * HT
            pltpu.sync_copy(p0_hbm.at[wid, pl.ds(hb, HT)], i0_v)
            pltpu.sync_copy(p1_hbm.at[wid, pl.ds(hb, HT)], i1_v)
            g0 = pltpu.async_copy(ys_hbm.at[i0_v], r0_v, s0)
            g1 = pltpu.async_copy(ys_hbm.at[i1_v], r1_v, s1)
            g0.wait()
            g1.wait()

            def row(i, _):
                c0t = c0_v[hb + i, :]
                c1t = c1_v[hb + i, :]
                for j in range(H // 16):
                    sl = pl.ds(16 * j, 16)
                    r0_v[i, sl] = c0t * r0_v[i, sl] + c1t * r1_v[i, sl]
                return 0

            lax.fori_loop(0, HT, row, 0)
            pltpu.sync_copy(r0_v, out_hbm.at[pl.ds(base + hb, HT)])

    return combine


def _sc_combine(ys, p0w, p1w, c0w, c1w):
    return _sc_combine_kernel()(ys, p0w, p1w, c0w, c1w)


def kernel(hidden_states, router_w, w_in, b_in, w_out, b_out):
    b, s, h = hidden_states.shape
    x = hidden_states.reshape(-1, h)
    logits, p0, p1, c0, c1, bexp, nvalid = _router(x, router_w)
    p0w = p0.reshape(NW, TPW)
    p1w = p1.reshape(NW, TPW)
    xs = _sc_dispatch(x, p0w, p1w)
    ys = _grouped(bexp.reshape(-1), nvalid.reshape(-1), xs, w_in, b_in, w_out, b_out)
    out = _sc_combine(
        ys, p0w, p1w, c0.reshape(NW, TPW, 16), c1.reshape(NW, TPW, 16)
    )
    return out.reshape(b, s, h), logits


# speedup vs baseline: 1.0132x; 1.0132x over previous
import functools

import jax
import jax.numpy as jnp
from jax import lax
from jax.experimental import pallas as pl
from jax.experimental.pallas import tpu as pltpu
from jax.experimental.pallas import tpu_sc as plsc

E = 8
H = 1024
I = 2048
T = 2048
K = 2

BLK = 288
NBMAX = (K * T) // BLK + E - 1
PMAX = NBMAX * BLK

NC = 2
NS = 16
NW = NC * NS
TPW = T // NW
HT = TPW // 2


def _dot_t(a, b):
    return jax.lax.dot_general(
        a, b, (((1,), (1,)), ((), ())), preferred_element_type=jnp.float32
    )


def _router_body(
    x_ref, rw_ref, logits_ref, p0_ref, p1_ref, c0_ref, c1_ref, bexp_ref, nvalid_ref
):
    x = x_ref[...]
    logits = _dot_t(x, rw_ref[...])
    logits_ref[...] = logits
    m = jnp.max(logits, axis=1, keepdims=True)
    ex = jnp.exp(logits - m)
    probs = ex / jnp.sum(ex, axis=1, keepdims=True)
    eidx = jax.lax.broadcasted_iota(jnp.int32, (T, E), 1)
    m1 = jnp.max(probs, axis=1, keepdims=True)
    i1 = jnp.min(jnp.where(probs == m1, eidx, E), axis=1, keepdims=True)
    probs2 = jnp.where(eidx == i1, -1.0, probs)
    m2 = jnp.max(probs2, axis=1, keepdims=True)
    i2 = jnp.min(jnp.where(probs2 == m2, eidx, E), axis=1, keepdims=True)
    denom = m1 + m2
    c0_ref[...] = jnp.broadcast_to(m1 / denom, (T, 16))
    c1_ref[...] = jnp.broadcast_to(m2 / denom, (T, 16))

    onehot0 = (eidx == i1).astype(jnp.float32)
    onehot1 = (eidx == i2).astype(jnp.float32)
    oh = jnp.concatenate([onehot0, onehot1], axis=1)
    CH = 512
    rr = jax.lax.broadcasted_iota(jnp.int32, (CH, CH), 0)
    cc = jax.lax.broadcasted_iota(jnp.int32, (CH, CH), 1)
    tric = (rr > cc).astype(jnp.float32)
    carry = jnp.zeros((1, 2 * E), jnp.float32)
    chunks = []
    for k in range(T // CH):
        ohk = oh[k * CH : (k + 1) * CH]
        chunks.append(
            jnp.dot(tric, ohk, preferred_element_type=jnp.float32) + carry
        )
        carry = carry + jnp.sum(ohk, axis=0, keepdims=True)
    pref = jnp.concatenate(chunks, axis=0)
    pref0 = pref[:, :E]
    pref1 = pref[:, E:]
    counts0 = carry[:, :E]
    counts = counts0 + carry[:, E:]
    padded = jnp.floor((counts + (BLK - 1)) * (1.0 / BLK)) * BLK
    eci = jax.lax.broadcasted_iota(jnp.int32, (E, E), 0)
    ecj = jax.lax.broadcasted_iota(jnp.int32, (E, E), 1)
    off = jnp.dot(
        padded, (eci < ecj).astype(jnp.float32), preferred_element_type=jnp.float32
    )
    nb = padded * (1.0 / BLK)
    cnb = jnp.dot(
        nb, (eci <= ecj).astype(jnp.float32), preferred_element_type=jnp.float32
    )

    def sel(tab, idx):
        return jnp.sum(jnp.where(eidx == idx, tab, 0.0), axis=1, keepdims=True)

    pos0 = sel(off, i1) + sel(pref0, i1)
    pos1 = sel(off, i2) + sel(counts0, i2) + sel(pref1, i2)
    p0_ref[...] = pos0.astype(jnp.int32)
    p1_ref[...] = pos1.astype(jnp.int32)

    bi = jax.lax.broadcasted_iota(jnp.int32, (1, NBMAX), 1).astype(jnp.float32)
    bexp = jnp.zeros((1, NBMAX), jnp.float32)
    for e in range(E):
        bexp += (bi >= cnb[0:1, e : e + 1]).astype(jnp.float32)
    bexp_ref[...] = bexp.astype(jnp.int32)
    nvalid_ref[...] = cnb[0:1, E - 1 : E].astype(jnp.int32)


def _router(x, router_w):
    return pl.pallas_call(
        _router_body,
        out_shape=(
            jax.ShapeDtypeStruct((T, E), jnp.float32),
            jax.ShapeDtypeStruct((T, 1), jnp.int32),
            jax.ShapeDtypeStruct((T, 1), jnp.int32),
            jax.ShapeDtypeStruct((T, 16), jnp.float32),
            jax.ShapeDtypeStruct((T, 16), jnp.float32),
            jax.ShapeDtypeStruct((1, NBMAX), jnp.int32),
            jax.ShapeDtypeStruct((1, 1), jnp.int32),
        ),
    )(x, router_w)


@functools.cache
def _sc_dispatch_kernel():
    mesh = plsc.VectorSubcoreMesh(
        core_axis_name="c", subcore_axis_name="s", num_cores=NC, num_subcores=NS
    )

    @functools.partial(
        pl.kernel,
        out_type=jax.ShapeDtypeStruct((PMAX, H), jnp.float32),
        mesh=mesh,
        scratch_types=[
            pltpu.VMEM((HT,), jnp.int32),
            pltpu.VMEM((HT,), jnp.int32),
            pltpu.VMEM((HT,), jnp.int32),
            pltpu.VMEM((HT,), jnp.int32),
            pltpu.VMEM((HT, H), jnp.float32),
            pltpu.VMEM((HT, H), jnp.float32),
            pltpu.SemaphoreType.DMA,
            pltpu.SemaphoreType.DMA,
        ],
    )
    def dispatch(
        x_hbm, p0_hbm, p1_hbm, xs_hbm,
        i0a_v, i1a_v, i0b_v, i1b_v, rows_a, rows_b, sem0, sem1,
    ):
        wid = lax.axis_index("s") * NC + lax.axis_index("c")
        base = wid * TPW
        pltpu.sync_copy(p0_hbm.at[wid, pl.ds(0, HT)], i0a_v)
        pltpu.sync_copy(p1_hbm.at[wid, pl.ds(0, HT)], i1a_v)
        pltpu.sync_copy(x_hbm.at[pl.ds(base, HT)], rows_a)
        ha0 = pltpu.async_copy(rows_a, xs_hbm.at[i0a_v], sem0)
        ha1 = pltpu.async_copy(rows_a, xs_hbm.at[i1a_v], sem1)
        pltpu.sync_copy(p0_hbm.at[wid, pl.ds(HT, HT)], i0b_v)
        pltpu.sync_copy(p1_hbm.at[wid, pl.ds(HT, HT)], i1b_v)
        pltpu.sync_copy(x_hbm.at[pl.ds(base + HT, HT)], rows_b)
        ha0.wait()
        ha1.wait()
        hb0 = pltpu.async_copy(rows_b, xs_hbm.at[i0b_v], sem0)
        hb1 = pltpu.async_copy(rows_b, xs_hbm.at[i1b_v], sem1)
        hb0.wait()
        hb1.wait()

    return dispatch


def _sc_dispatch(x, p0w, p1w):
    return _sc_dispatch_kernel()(x, p0w, p1w)


def _grouped_body(
    bexp_s, nvalid_s, xs_ref, w_in_ref, b_in_ref, w_out_ref, b_out_ref, out_ref
):
    i = pl.program_id(0)
    e = jnp.minimum(bexp_s[i], E - 1)

    @pl.when(i < nvalid_s[0])
    def _():
        x = xs_ref[...]
        mid = _dot_t(x, w_in_ref[0])
        mid = mid + b_in_ref[e, :][None, :]
        mid = 0.5 * mid * (1.0 + jax.lax.erf(mid * 0.7071067811865476))
        y = _dot_t(mid, w_out_ref[0])
        out_ref[...] = y + b_out_ref[e, :][None, :]


def _grouped(bexp, nvalid, xs, w_in, b_in, w_out, b_out):
    def emap(i, bexp_s, nvalid_s):
        return (jnp.minimum(bexp_s[i], E - 1), 0, 0)

    grid_spec = pltpu.PrefetchScalarGridSpec(
        num_scalar_prefetch=2,
        grid=(NBMAX,),
        in_specs=[
            pl.BlockSpec((BLK, H), lambda i, b, n: (i, 0)),
            pl.BlockSpec((1, I, H), emap),
            pl.BlockSpec((E, I), lambda i, b, n: (0, 0)),
            pl.BlockSpec((1, H, I), emap),
            pl.BlockSpec((E, H), lambda i, b, n: (0, 0)),
        ],
        out_specs=pl.BlockSpec((BLK, H), lambda i, b, n: (i, 0)),
    )
    return pl.pallas_call(
        _grouped_body,
        grid_spec=grid_spec,
        out_shape=jax.ShapeDtypeStruct((PMAX, H), jnp.float32),
        compiler_params=pltpu.CompilerParams(
            dimension_semantics=("arbitrary",),
        ),
    )(bexp, nvalid, xs, w_in, b_in, w_out, b_out)


@functools.cache
def _sc_combine_kernel():
    mesh = plsc.VectorSubcoreMesh(
        core_axis_name="c", subcore_axis_name="s", num_cores=NC, num_subcores=NS
    )

    @functools.partial(
        pl.kernel,
        out_type=jax.ShapeDtypeStruct((T, H), jnp.float32),
        mesh=mesh,
        scratch_types=[
            pltpu.VMEM((HT,), jnp.int32),
            pltpu.VMEM((HT,), jnp.int32),
            pltpu.VMEM((HT, H), jnp.float32),
            pltpu.VMEM((HT, H), jnp.float32),
            pltpu.VMEM((TPW, 16), jnp.float32),
            pltpu.VMEM((TPW, 16), jnp.float32),
            pltpu.SemaphoreType.DMA,
            pltpu.SemaphoreType.DMA,
        ],
    )
    def combine(
        ys_hbm, p0_hbm, p1_hbm, c0_hbm, c1_hbm, out_hbm,
        i0_v, i1_v, r0_v, r1_v, c0_v, c1_v, s0, s1,
    ):
        wid = lax.axis_index("s") * NC + lax.axis_index("c")
        base = wid * TPW
        pltpu.sync_copy(c0_hbm.at[wid], c0_v)
        pltpu.sync_copy(c1_hbm.at[wid], c1_v)
        for h in range(TPW // HT):
            hb = h * HT
            pltpu.sync_copy(p0_hbm.at[wid, pl.ds(hb, HT)], i0_v)
            pltpu.sync_copy(p1_hbm.at[wid, pl.ds(hb, HT)], i1_v)
            g0 = pltpu.async_copy(ys_hbm.at[i0_v], r0_v, s0)
            g1 = pltpu.async_copy(ys_hbm.at[i1_v], r1_v, s1)
            g0.wait()
            g1.wait()

            def row(i, _):
                c0t = c0_v[hb + i, :]
                c1t = c1_v[hb + i, :]
                for j in range(H // 16):
                    sl = pl.ds(16 * j, 16)
                    r0_v[i, sl] = c0t * r0_v[i, sl] + c1t * r1_v[i, sl]
                return 0

            lax.fori_loop(0, HT, row, 0)
            pltpu.sync_copy(r0_v, out_hbm.at[pl.ds(base + hb, HT)])

    return combine


def _sc_combine(ys, p0w, p1w, c0w, c1w):
    return _sc_combine_kernel()(ys, p0w, p1w, c0w, c1w)


def kernel(hidden_states, router_w, w_in, b_in, w_out, b_out):
    b, s, h = hidden_states.shape
    x = hidden_states.reshape(-1, h)
    logits, p0, p1, c0, c1, bexp, nvalid = _router(x, router_w)
    p0w = p0.reshape(NW, TPW)
    p1w = p1.reshape(NW, TPW)
    xs = _sc_dispatch(x, p0w, p1w)
    ys = _grouped(bexp.reshape(-1), nvalid.reshape(-1), xs, w_in, b_in, w_out, b_out)
    out = _sc_combine(
        ys, p0w, p1w, c0.reshape(NW, TPW, 16), c1.reshape(NW, TPW, 16)
    )
    return out.reshape(b, s, h), logits


# speedup vs baseline: 1.0248x; 1.0115x over previous
import functools

import jax
import jax.numpy as jnp
from jax import lax
from jax.experimental import pallas as pl
from jax.experimental.pallas import tpu as pltpu
from jax.experimental.pallas import tpu_sc as plsc

E = 8
H = 1024
I = 2048
T = 2048
K = 2

BLK = 288
NBMAX = (K * T) // BLK + E - 1
PMAX = NBMAX * BLK

NC = 2
NS = 16
NW = NC * NS
TPW = T // NW
HT = TPW // 2


def _dot_t(a, b):
    return jax.lax.dot_general(
        a, b, (((1,), (1,)), ((), ())), preferred_element_type=jnp.float32
    )


def _router_body(
    x_ref, rw_ref, logits_ref, p0_ref, p1_ref, c0_ref, c1_ref, bexp_ref, nvalid_ref
):
    x = x_ref[...]
    logits = _dot_t(x, rw_ref[...])
    logits_ref[...] = logits
    m = jnp.max(logits, axis=1, keepdims=True)
    ex = jnp.exp(logits - m)
    probs = ex / jnp.sum(ex, axis=1, keepdims=True)
    eidx = jax.lax.broadcasted_iota(jnp.int32, (T, E), 1)
    m1 = jnp.max(probs, axis=1, keepdims=True)
    i1 = jnp.min(jnp.where(probs == m1, eidx, E), axis=1, keepdims=True)
    probs2 = jnp.where(eidx == i1, -1.0, probs)
    m2 = jnp.max(probs2, axis=1, keepdims=True)
    i2 = jnp.min(jnp.where(probs2 == m2, eidx, E), axis=1, keepdims=True)
    denom = m1 + m2
    c0_ref[...] = jnp.broadcast_to(m1 / denom, (T, 16))
    c1_ref[...] = jnp.broadcast_to(m2 / denom, (T, 16))

    onehot0 = (eidx == i1).astype(jnp.float32)
    onehot1 = (eidx == i2).astype(jnp.float32)
    oh = jnp.concatenate([onehot0, onehot1], axis=1)
    CH = 512
    rr = jax.lax.broadcasted_iota(jnp.int32, (CH, CH), 0)
    cc = jax.lax.broadcasted_iota(jnp.int32, (CH, CH), 1)
    tric = (rr > cc).astype(jnp.float32)
    carry = jnp.zeros((1, 2 * E), jnp.float32)
    chunks = []
    for k in range(T // CH):
        ohk = oh[k * CH : (k + 1) * CH]
        chunks.append(
            jnp.dot(tric, ohk, preferred_element_type=jnp.float32) + carry
        )
        carry = carry + jnp.sum(ohk, axis=0, keepdims=True)
    pref = jnp.concatenate(chunks, axis=0)
    pref0 = pref[:, :E]
    pref1 = pref[:, E:]
    counts0 = carry[:, :E]
    counts = counts0 + carry[:, E:]
    padded = jnp.floor((counts + (BLK - 1)) * (1.0 / BLK)) * BLK
    eci = jax.lax.broadcasted_iota(jnp.int32, (E, E), 0)
    ecj = jax.lax.broadcasted_iota(jnp.int32, (E, E), 1)
    off = jnp.dot(
        padded, (eci < ecj).astype(jnp.float32), preferred_element_type=jnp.float32
    )
    nb = padded * (1.0 / BLK)
    cnb = jnp.dot(
        nb, (eci <= ecj).astype(jnp.float32), preferred_element_type=jnp.float32
    )

    def sel(tab, idx):
        return jnp.sum(jnp.where(eidx == idx, tab, 0.0), axis=1, keepdims=True)

    pos0 = sel(off, i1) + sel(pref0, i1)
    pos1 = sel(off, i2) + sel(counts0, i2) + sel(pref1, i2)
    p0_ref[...] = pos0.astype(jnp.int32)
    p1_ref[...] = pos1.astype(jnp.int32)

    bi = jax.lax.broadcasted_iota(jnp.int32, (1, NBMAX), 1).astype(jnp.float32)
    bexp = jnp.zeros((1, NBMAX), jnp.float32)
    for e in range(E):
        bexp += (bi >= cnb[0:1, e : e + 1]).astype(jnp.float32)
    bexp_ref[...] = bexp.astype(jnp.int32)
    nvalid_ref[...] = cnb[0:1, E - 1 : E].astype(jnp.int32)


def _router(x, router_w):
    return pl.pallas_call(
        _router_body,
        out_shape=(
            jax.ShapeDtypeStruct((T, E), jnp.float32),
            jax.ShapeDtypeStruct((T, 1), jnp.int32),
            jax.ShapeDtypeStruct((T, 1), jnp.int32),
            jax.ShapeDtypeStruct((T, 16), jnp.float32),
            jax.ShapeDtypeStruct((T, 16), jnp.float32),
            jax.ShapeDtypeStruct((1, NBMAX), jnp.int32),
            jax.ShapeDtypeStruct((1, 1), jnp.int32),
        ),
    )(x, router_w)


@functools.cache
def _sc_dispatch_kernel():
    mesh = plsc.VectorSubcoreMesh(
        core_axis_name="c", subcore_axis_name="s", num_cores=NC, num_subcores=NS
    )

    @functools.partial(
        pl.kernel,
        out_type=jax.ShapeDtypeStruct((PMAX, H), jnp.float32),
        mesh=mesh,
        scratch_types=[
            pltpu.VMEM((TPW,), jnp.int32),
            pltpu.VMEM((TPW,), jnp.int32),
            pltpu.VMEM((TPW, H), jnp.float32),
            pltpu.SemaphoreType.DMA,
            pltpu.SemaphoreType.DMA,
        ],
    )
    def dispatch(x_hbm, p0_hbm, p1_hbm, xs_hbm, idx0_v, idx1_v, rows_v, sem0, sem1):
        wid = lax.axis_index("s") * NC + lax.axis_index("c")
        base = wid * TPW
        pltpu.sync_copy(p0_hbm.at[wid], idx0_v)
        pltpu.sync_copy(p1_hbm.at[wid], idx1_v)
        pltpu.sync_copy(x_hbm.at[pl.ds(base, TPW)], rows_v)
        h0 = pltpu.async_copy(rows_v, xs_hbm.at[idx0_v], sem0)
        h1 = pltpu.async_copy(rows_v, xs_hbm.at[idx1_v], sem1)
        h0.wait()
        h1.wait()

    return dispatch


def _sc_dispatch(x, p0w, p1w):
    return _sc_dispatch_kernel()(x, p0w, p1w)


def _grouped_body(
    bexp_s, nvalid_s, xs_ref, w_in_ref, b_in_ref, w_out_ref, b_out_ref, out_ref
):
    i = pl.program_id(0)
    e = jnp.minimum(bexp_s[i], E - 1)

    @pl.when(i < nvalid_s[0])
    def _():
        x = xs_ref[...]
        mid = _dot_t(x, w_in_ref[0])
        mid = mid + b_in_ref[e, :][None, :]
        mid = 0.5 * mid * (1.0 + jax.lax.erf(mid * 0.7071067811865476))
        y = _dot_t(mid, w_out_ref[0])
        out_ref[...] = y + b_out_ref[e, :][None, :]


def _grouped(bexp, nvalid, xs, w_in, b_in, w_out, b_out):
    def emap(i, bexp_s, nvalid_s):
        return (jnp.minimum(bexp_s[i], E - 1), 0, 0)

    grid_spec = pltpu.PrefetchScalarGridSpec(
        num_scalar_prefetch=2,
        grid=(NBMAX,),
        in_specs=[
            pl.BlockSpec((BLK, H), lambda i, b, n: (i, 0)),
            pl.BlockSpec((1, I, H), emap),
            pl.BlockSpec((E, I), lambda i, b, n: (0, 0)),
            pl.BlockSpec((1, H, I), emap),
            pl.BlockSpec((E, H), lambda i, b, n: (0, 0)),
        ],
        out_specs=pl.BlockSpec((BLK, H), lambda i, b, n: (i, 0)),
    )
    return pl.pallas_call(
        _grouped_body,
        grid_spec=grid_spec,
        out_shape=jax.ShapeDtypeStruct((PMAX, H), jnp.float32),
        compiler_params=pltpu.CompilerParams(
            dimension_semantics=("arbitrary",),
        ),
    )(bexp, nvalid, xs, w_in, b_in, w_out, b_out)


@functools.cache
def _sc_combine_kernel():
    mesh = plsc.VectorSubcoreMesh(
        core_axis_name="c", subcore_axis_name="s", num_cores=NC, num_subcores=NS
    )

    @functools.partial(
        pl.kernel,
        out_type=jax.ShapeDtypeStruct((T, H), jnp.float32),
        mesh=mesh,
        scratch_types=[
            pltpu.VMEM((HT,), jnp.int32),
            pltpu.VMEM((HT,), jnp.int32),
            pltpu.VMEM((HT, H), jnp.float32),
            pltpu.VMEM((HT, H), jnp.float32),
            pltpu.VMEM((TPW, 16), jnp.float32),
            pltpu.VMEM((TPW, 16), jnp.float32),
            pltpu.SemaphoreType.DMA,
            pltpu.SemaphoreType.DMA,
        ],
    )
    def combine(
        ys_hbm, p0_hbm, p1_hbm, c0_hbm, c1_hbm, out_hbm,
        i0_v, i1_v, r0_v, r1_v, c0_v, c1_v, s0, s1,
    ):
        wid = lax.axis_index("s") * NC + lax.axis_index("c")
        base = wid * TPW
        pltpu.sync_copy(c0_hbm.at[wid], c0_v)
        pltpu.sync_copy(c1_hbm.at[wid], c1_v)
        for h in range(TPW // HT):
            hb = h * HT
            pltpu.sync_copy(p0_hbm.at[wid, pl.ds(hb, HT)], i0_v)
            pltpu.sync_copy(p1_hbm.at[wid, pl.ds(hb, HT)], i1_v)
            g0 = pltpu.async_copy(ys_hbm.at[i0_v], r0_v, s0)
            g1 = pltpu.async_copy(ys_hbm.at[i1_v], r1_v, s1)
            g0.wait()
            g1.wait()

            def row(i, _):
                c0t = c0_v[hb + i, :]
                c1t = c1_v[hb + i, :]
                for j in range(H // 16):
                    sl = pl.ds(16 * j, 16)
                    r0_v[i, sl] = c0t * r0_v[i, sl] + c1t * r1_v[i, sl]
                return 0

            lax.fori_loop(0, HT, row, 0)
            pltpu.sync_copy(r0_v, out_hbm.at[pl.ds(base + hb, HT)])

    return combine


def _sc_combine(ys, p0w, p1w, c0w, c1w):
    return _sc_combine_kernel()(ys, p0w, p1w, c0w, c1w)


def kernel(hidden_states, router_w, w_in, b_in, w_out, b_out):
    b, s, h = hidden_states.shape
    x = hidden_states.reshape(-1, h)
    logits, p0, p1, c0, c1, bexp, nvalid = _router(x, router_w)
    p0w = p0.reshape(NW, TPW)
    p1w = p1.reshape(NW, TPW)
    xs = _sc_dispatch(x, p0w, p1w)
    ys = _grouped(bexp.reshape(-1), nvalid.reshape(-1), xs, w_in, b_in, w_out, b_out)
    out = _sc_combine(
        ys, p0w, p1w, c0.reshape(NW, TPW, 16), c1.reshape(NW, TPW, 16)
    )
    return out.reshape(b, s, h), logits
